# baseline (speedup 1.0000x reference)
import dataclasses
import functools

import jax
import jax.numpy as jnp
from jax import lax
from jax.experimental import pallas as pl
from jax.experimental.pallas import tpu as pltpu
from jax.experimental.pallas import tpu_sc as plsc

NUM_RADIAL = 16
HIDDEN = 256
INT_EMB = 128
N_NODES = 10000
N_EDGES = 160000

HW = INT_EMB
NC, NS = 2, 16
NW = NC * NS
CHUNK = 128
CPW = 40
PAIRS = CPW // 2
E_PAD = NW * CPW * CHUNK
N_PAD = 10240

BE = 1280
BN = 2000


def _h_body(ea_ref, w1t_ref, b1_ref, h_ref):
    a = jnp.dot(ea_ref[...], w1t_ref[...], preferred_element_type=jnp.float32)
    a = a + b1_ref[...]
    h_ref[...] = a * jax.nn.sigmoid(a)


def _out_body(acc_ref, deg_ref, x_ref, w2t_ref, b2_ref, o_ref):
    nh = acc_ref[0] + acc_ref[1]
    deg = jnp.sum(deg_ref[...], axis=1, keepdims=True)
    o_ref[...] = (x_ref[...]
                  + jnp.dot(nh, w2t_ref[...], preferred_element_type=jnp.float32)
                  + deg * b2_ref[...])


_vmesh = plsc.VectorSubcoreMesh(core_axis_name="c", subcore_axis_name="s")

_sc_params = pltpu.CompilerParams()
if "needs_layout_passes" in pltpu.CompilerParams.__dataclass_fields__:
    _sc_params = dataclasses.replace(_sc_params, needs_layout_passes=False)


@functools.partial(
    pl.kernel,
    out_type=(
        jax.ShapeDtypeStruct((NC, N_NODES, HW), jnp.float32),
        jax.ShapeDtypeStruct((NW * N_NODES,), jnp.float32),
    ),
    mesh=_vmesh,
    compiler_params=_sc_params,
    scratch_types=[
        pltpu.VMEM((CPW, CHUNK), jnp.int32),
        pltpu.VMEM((CHUNK, HW), jnp.float32),
        pltpu.VMEM((CHUNK, HW), jnp.float32),
        pltpu.VMEM((N_PAD,), jnp.float32),
        pltpu.VMEM_SHARED((N_PAD, HW), jnp.float32),
        pltpu.SemaphoreType.DMA,
        pltpu.SemaphoreType.DMA,
    ],
)
def _scatter_kernel(h_hbm, idx_hbm, zero_hbm, out_hbm, deg_hbm,
                    idx_v, h_a, h_b, deg_v, acc_sh, sem_a, sem_b):
    c = lax.axis_index("c")
    s = lax.axis_index("s")
    wid = c * NS + s

    pltpu.sync_copy(zero_hbm.at[pl.ds(s * 640, 640)],
                    acc_sh.at[pl.ds(s * 640, 640)])

    zeros16 = jnp.zeros((16,), jnp.float32)
    @pl.loop(0, N_PAD // 16)
    def _(i):
        deg_v[pl.ds(i * 16, 16)] = zeros16

    plsc.subcore_barrier()

    base_chunk = wid * CPW
    e_base = base_chunk * CHUNK
    pltpu.sync_copy(idx_hbm.at[pl.ds(base_chunk, CPW)], idx_v)

    ones16 = jnp.ones((16,), jnp.float32)

    def deg_update(j):
        @pl.loop(0, CHUNK // 16)
        def _(k):
            idx16 = idx_v[j, pl.ds(k * 16, 16)]
            plsc.addupdate_scatter(deg_v, [idx16], ones16)

    def load(j, buf, sem):
        pltpu.make_async_copy(
            h_hbm.at[pl.ds(e_base + j * CHUNK, CHUNK)], buf, sem).start()

    def drain_load(buf, sem):
        pltpu.make_async_copy(h_hbm.at[pl.ds(0, CHUNK)], buf, sem).wait()

    load(0, h_a, sem_a)

    @pl.loop(0, PAIRS)
    def _(t):
        c0 = 2 * t
        c1 = c0 + 1
        load(c1, h_b, sem_b)
        drain_load(h_a, sem_a)
        sc_a = pltpu.async_copy(h_a, acc_sh.at[idx_v.at[c0]], sem_a, add=True)
        deg_update(c0)
        drain_load(h_b, sem_b)
        sc_a.wait()

        @pl.when(t < PAIRS - 1)
        def _():
            load(c0 + 2, h_a, sem_a)

        sc_b = pltpu.async_copy(h_b, acc_sh.at[idx_v.at[c1]], sem_b, add=True)
        deg_update(c1)
        sc_b.wait()

    pltpu.sync_copy(deg_v.at[pl.ds(0, N_NODES)],
                    deg_hbm.at[pl.ds(wid * N_NODES, N_NODES)])

    plsc.subcore_barrier()

    @pl.when(s < 10)
    def _():
        pltpu.sync_copy(acc_sh.at[pl.ds(s * 1000, 1000)],
                        out_hbm.at[c].at[pl.ds(s * 1000, 1000)])


def kernel(x, edge_index, edge_attr, W1, b1, W2, b2):
    col = edge_index[1].astype(jnp.int32)
    col_pad = jnp.concatenate(
        [col, jnp.full((E_PAD - N_EDGES,), N_NODES, jnp.int32)])
    idx2d = col_pad.reshape(NW * CPW, CHUNK)

    w1t = W1.T
    b1r = b1.reshape(1, INT_EMB)
    w2t = W2.T
    b2r = b2.reshape(1, HIDDEN)

    n_real_blocks = N_EDGES // BE
    h = pl.pallas_call(
        _h_body,
        grid=(E_PAD // BE,),
        in_specs=[
            pl.BlockSpec((BE, NUM_RADIAL),
                         lambda i: (jnp.minimum(i, n_real_blocks - 1), 0)),
            pl.BlockSpec((NUM_RADIAL, INT_EMB), lambda i: (0, 0)),
            pl.BlockSpec((1, INT_EMB), lambda i: (0, 0)),
        ],
        out_specs=pl.BlockSpec((BE, HW), lambda i: (i, 0)),
        out_shape=jax.ShapeDtypeStruct((E_PAD, HW), jnp.float32),
        compiler_params=pltpu.CompilerParams(
            dimension_semantics=("parallel",)),
    )(edge_attr, w1t, b1r)

    zero = jnp.zeros((N_PAD, HW), jnp.float32)
    acc, deg = _scatter_kernel(h, idx2d, zero)
    deg2d = deg.reshape(NW, N_NODES).T

    out = pl.pallas_call(
        _out_body,
        grid=(N_NODES // BN,),
        in_specs=[
            pl.BlockSpec((NC, BN, HW), lambda i: (0, i, 0)),
            pl.BlockSpec((BN, NW), lambda i: (i, 0)),
            pl.BlockSpec((BN, HIDDEN), lambda i: (i, 0)),
            pl.BlockSpec((INT_EMB, HIDDEN), lambda i: (0, 0)),
            pl.BlockSpec((1, HIDDEN), lambda i: (0, 0)),
        ],
        out_specs=pl.BlockSpec((BN, HIDDEN), lambda i: (i, 0)),
        out_shape=jax.ShapeDtypeStruct((N_NODES, HIDDEN), jnp.float32),
        compiler_params=pltpu.CompilerParams(
            dimension_semantics=("parallel",)),
    )(acc, deg2d, x, w2t, b2r)
    return out


# speedup vs baseline: 1.2711x; 1.2711x over previous
import dataclasses
import functools

import jax
import jax.numpy as jnp
from jax import lax
from jax.experimental import pallas as pl
from jax.experimental.pallas import tpu as pltpu
from jax.experimental.pallas import tpu_sc as plsc

NUM_RADIAL = 16
HIDDEN = 256
INT_EMB = 128
N_NODES = 10000
N_EDGES = 160000

HW = INT_EMB
NC, NS = 2, 16
NW = NC * NS
CHUNK = 128
CPW = 40
PAIRS = CPW // 2
E_PAD = NW * CPW * CHUNK
N_PAD = 10240

BE = 1280
BN = 2000


def _h_body(eat_ref, w1t_ref, b1_ref, h_ref):
    a = jax.lax.dot_general(
        eat_ref[...], w1t_ref[...], (((0,), (0,)), ((), ())),
        preferred_element_type=jnp.float32)
    a = a + b1_ref[...]
    h_ref[...] = a * jax.nn.sigmoid(a)


def _out_body(acc_ref, deg_ref, x_ref, w2t_ref, b2_ref, o_ref):
    nh = acc_ref[0] + acc_ref[1]
    deg = jnp.sum(deg_ref[...], axis=1, keepdims=True)
    o_ref[...] = (x_ref[...]
                  + jnp.dot(nh, w2t_ref[...], preferred_element_type=jnp.float32)
                  + deg * b2_ref[...])


_vmesh = plsc.VectorSubcoreMesh(core_axis_name="c", subcore_axis_name="s")

_sc_params = pltpu.CompilerParams()
if "needs_layout_passes" in pltpu.CompilerParams.__dataclass_fields__:
    _sc_params = dataclasses.replace(_sc_params, needs_layout_passes=False)


@functools.partial(
    pl.kernel,
    out_type=(
        jax.ShapeDtypeStruct((NC, N_NODES, HW), jnp.float32),
        jax.ShapeDtypeStruct((NW * N_NODES,), jnp.float32),
    ),
    mesh=_vmesh,
    compiler_params=_sc_params,
    scratch_types=[
        pltpu.VMEM((CPW, CHUNK), jnp.int32),
        pltpu.VMEM((CHUNK, HW), jnp.float32),
        pltpu.VMEM((CHUNK, HW), jnp.float32),
        pltpu.VMEM((N_PAD,), jnp.float32),
        pltpu.VMEM_SHARED((N_PAD, HW), jnp.float32),
        pltpu.SemaphoreType.DMA,
        pltpu.SemaphoreType.DMA,
    ],
)
def _scatter_kernel(h_hbm, idx_hbm, zero_hbm, out_hbm, deg_hbm,
                    idx_v, h_a, h_b, deg_v, acc_sh, sem_a, sem_b):
    c = lax.axis_index("c")
    s = lax.axis_index("s")
    wid = c * NS + s

    pltpu.sync_copy(zero_hbm.at[pl.ds(s * 640, 640)],
                    acc_sh.at[pl.ds(s * 640, 640)])

    zeros16 = jnp.zeros((16,), jnp.float32)
    @pl.loop(0, N_PAD // 16)
    def _(i):
        deg_v[pl.ds(i * 16, 16)] = zeros16

    plsc.subcore_barrier()

    base_chunk = wid * CPW
    e_base = base_chunk * CHUNK
    pltpu.sync_copy(idx_hbm.at[pl.ds(base_chunk, CPW)], idx_v)

    ones16 = jnp.ones((16,), jnp.float32)

    def deg_update(j):
        @pl.loop(0, CHUNK // 16)
        def _(k):
            idx16 = idx_v[j, pl.ds(k * 16, 16)]
            plsc.addupdate_scatter(deg_v, [idx16], ones16)

    def load(j, buf, sem):
        pltpu.make_async_copy(
            h_hbm.at[pl.ds(e_base + j * CHUNK, CHUNK)], buf, sem).start()

    def drain_load(buf, sem):
        pltpu.make_async_copy(h_hbm.at[pl.ds(0, CHUNK)], buf, sem).wait()

    load(0, h_a, sem_a)

    @pl.loop(0, PAIRS)
    def _(t):
        c0 = 2 * t
        c1 = c0 + 1
        load(c1, h_b, sem_b)
        drain_load(h_a, sem_a)
        sc_a = pltpu.async_copy(h_a, acc_sh.at[idx_v.at[c0]], sem_a, add=True)
        deg_update(c0)
        drain_load(h_b, sem_b)
        sc_a.wait()

        @pl.when(t < PAIRS - 1)
        def _():
            load(c0 + 2, h_a, sem_a)

        sc_b = pltpu.async_copy(h_b, acc_sh.at[idx_v.at[c1]], sem_b, add=True)
        deg_update(c1)
        sc_b.wait()

    pltpu.sync_copy(deg_v.at[pl.ds(0, N_NODES)],
                    deg_hbm.at[pl.ds(wid * N_NODES, N_NODES)])

    plsc.subcore_barrier()

    @pl.when(s < 10)
    def _():
        pltpu.sync_copy(acc_sh.at[pl.ds(s * 1000, 1000)],
                        out_hbm.at[c].at[pl.ds(s * 1000, 1000)])


def kernel(x, edge_index, edge_attr, W1, b1, W2, b2):
    col = edge_index[1].astype(jnp.int32)
    col_pad = jnp.concatenate(
        [col, jnp.full((E_PAD - N_EDGES,), N_NODES, jnp.int32)])
    idx2d = col_pad.reshape(NW * CPW, CHUNK)

    w1t = W1.T
    b1r = b1.reshape(1, INT_EMB)
    w2t = W2.T
    b2r = b2.reshape(1, HIDDEN)

    n_real_blocks = N_EDGES // BE
    ea_t = edge_attr.T
    h = pl.pallas_call(
        _h_body,
        grid=(E_PAD // BE,),
        in_specs=[
            pl.BlockSpec((NUM_RADIAL, BE),
                         lambda i: (0, jnp.minimum(i, n_real_blocks - 1))),
            pl.BlockSpec((NUM_RADIAL, INT_EMB), lambda i: (0, 0)),
            pl.BlockSpec((1, INT_EMB), lambda i: (0, 0)),
        ],
        out_specs=pl.BlockSpec((BE, HW), lambda i: (i, 0)),
        out_shape=jax.ShapeDtypeStruct((E_PAD, HW), jnp.float32),
        compiler_params=pltpu.CompilerParams(
            dimension_semantics=("parallel",)),
    )(ea_t, w1t, b1r)

    zero = jnp.zeros((N_PAD, HW), jnp.float32)
    acc, deg = _scatter_kernel(h, idx2d, zero)
    deg2d = deg.reshape(NW, N_NODES).T

    out = pl.pallas_call(
        _out_body,
        grid=(N_NODES // BN,),
        in_specs=[
            pl.BlockSpec((NC, BN, HW), lambda i: (0, i, 0)),
            pl.BlockSpec((BN, NW), lambda i: (i, 0)),
            pl.BlockSpec((BN, HIDDEN), lambda i: (i, 0)),
            pl.BlockSpec((INT_EMB, HIDDEN), lambda i: (0, 0)),
            pl.BlockSpec((1, HIDDEN), lambda i: (0, 0)),
        ],
        out_specs=pl.BlockSpec((BN, HIDDEN), lambda i: (i, 0)),
        out_shape=jax.ShapeDtypeStruct((N_NODES, HIDDEN), jnp.float32),
        compiler_params=pltpu.CompilerParams(
            dimension_semantics=("parallel",)),
    )(acc, deg2d, x, w2t, b2r)
    return out


# speedup vs baseline: 1.7603x; 1.3849x over previous
import dataclasses
import functools

import jax
import jax.numpy as jnp
from jax import lax
from jax.experimental import pallas as pl
from jax.experimental.pallas import tpu as pltpu
from jax.experimental.pallas import tpu_sc as plsc

NUM_RADIAL = 16
HIDDEN = 256
INT_EMB = 128
N_NODES = 10000
N_EDGES = 160000

HW = INT_EMB
NC, NS = 2, 16
NW = NC * NS
CHUNK = 128
N_CHUNKS = N_EDGES // CHUNK
CPW = 40
PAIRS = CPW // 2
N_PAD = 10240

BE = 6400
BN = 2000


def _h_body(eat_ref, w1t_ref, b1_ref, h_ref):
    a = jax.lax.dot_general(
        eat_ref[...], w1t_ref[...], (((0,), (0,)), ((), ())),
        preferred_element_type=jnp.float32)
    a = a + b1_ref[...]
    h_ref[...] = a * jax.nn.sigmoid(a)


def _out_body(acc_ref, deg_ref, x_ref, w2t_ref, b2_ref, o_ref):
    nh = acc_ref[0] + acc_ref[1]
    deg = jnp.sum(deg_ref[...], axis=1, keepdims=True)
    o_ref[...] = (x_ref[...]
                  + jnp.dot(nh, w2t_ref[...], preferred_element_type=jnp.float32)
                  + deg * b2_ref[...])


_vmesh = plsc.VectorSubcoreMesh(core_axis_name="c", subcore_axis_name="s")

_sc_params = pltpu.CompilerParams()
if "needs_layout_passes" in pltpu.CompilerParams.__dataclass_fields__:
    _sc_params = dataclasses.replace(_sc_params, needs_layout_passes=False)


@functools.partial(
    pl.kernel,
    out_type=(
        jax.ShapeDtypeStruct((NC, N_NODES, HW), jnp.float32),
        jax.ShapeDtypeStruct((NW * N_NODES,), jnp.float32),
    ),
    mesh=_vmesh,
    compiler_params=_sc_params,
    scratch_types=[
        pltpu.VMEM((CPW, CHUNK), jnp.int32),
        pltpu.VMEM((CHUNK, HW), jnp.float32),
        pltpu.VMEM((CHUNK, HW), jnp.float32),
        pltpu.VMEM((N_PAD,), jnp.float32),
        pltpu.VMEM_SHARED((N_PAD, HW), jnp.float32),
        pltpu.SemaphoreType.DMA,
        pltpu.SemaphoreType.DMA,
    ],
)
def _scatter_kernel(h_hbm, idx_hbm, zero_hbm, out_hbm, deg_hbm,
                    idx_v, h_a, h_b, deg_v, acc_sh, sem_a, sem_b):
    c = lax.axis_index("c")
    s = lax.axis_index("s")
    wid = c * NS + s

    pltpu.sync_copy(zero_hbm.at[pl.ds(s * 640, 640)],
                    acc_sh.at[pl.ds(s * 640, 640)])

    zeros16 = jnp.zeros((16,), jnp.float32)
    @pl.loop(0, N_PAD // 16)
    def _(i):
        deg_v[pl.ds(i * 16, 16)] = zeros16

    plsc.subcore_barrier()

    base_chunk = wid * CPW
    e_base = base_chunk * CHUNK
    n_live = jnp.minimum(CPW, N_CHUNKS - base_chunk)
    pltpu.sync_copy(idx_hbm.at[pl.ds(base_chunk, CPW)], idx_v)

    ones16 = jnp.ones((16,), jnp.float32)

    def deg_update(j):
        @pl.loop(0, CHUNK // 16)
        def _(k):
            idx16 = idx_v[j, pl.ds(k * 16, 16)]
            plsc.addupdate_scatter(deg_v, [idx16], ones16)

    def load(j, buf, sem):
        pltpu.make_async_copy(
            h_hbm.at[pl.ds(e_base + j * CHUNK, CHUNK)], buf, sem).start()

    def drain_load(buf, sem):
        pltpu.make_async_copy(h_hbm.at[pl.ds(0, CHUNK)], buf, sem).wait()

    @pl.when(n_live > 0)
    def _():
        load(0, h_a, sem_a)

    @pl.loop(0, PAIRS)
    def _(t):
        c0 = 2 * t
        c1 = c0 + 1

        @pl.when(c0 < n_live)
        def _():
            @pl.when(c1 < n_live)
            def _():
                load(c1, h_b, sem_b)
            drain_load(h_a, sem_a)
            sc_a = pltpu.async_copy(h_a, acc_sh.at[idx_v.at[c0]], sem_a,
                                    add=True)
            deg_update(c0)
            sc_a.wait()

            @pl.when(c1 < n_live)
            def _():
                drain_load(h_b, sem_b)

                @pl.when(c0 + 2 < n_live)
                def _():
                    load(c0 + 2, h_a, sem_a)

                sc_b = pltpu.async_copy(h_b, acc_sh.at[idx_v.at[c1]], sem_b,
                                        add=True)
                deg_update(c1)
                sc_b.wait()

    pltpu.sync_copy(deg_v.at[pl.ds(0, N_NODES)],
                    deg_hbm.at[pl.ds(wid * N_NODES, N_NODES)])

    plsc.subcore_barrier()

    @pl.when(s < 10)
    def _():
        pltpu.sync_copy(acc_sh.at[pl.ds(s * 1000, 1000)],
                        out_hbm.at[c].at[pl.ds(s * 1000, 1000)])


def kernel(x, edge_index, edge_attr, W1, b1, W2, b2):
    col = edge_index[1].astype(jnp.int32)
    col_pad = jnp.concatenate(
        [col, jnp.zeros((NW * CPW * CHUNK - N_EDGES,), jnp.int32)])
    idx2d = col_pad.reshape(NW * CPW, CHUNK)

    w1t = W1.T
    b1r = b1.reshape(1, INT_EMB)
    w2t = W2.T
    b2r = b2.reshape(1, HIDDEN)

    ea_t = edge_attr.T
    h = pl.pallas_call(
        _h_body,
        grid=(N_EDGES // BE,),
        in_specs=[
            pl.BlockSpec((NUM_RADIAL, BE), lambda i: (0, i)),
            pl.BlockSpec((NUM_RADIAL, INT_EMB), lambda i: (0, 0)),
            pl.BlockSpec((1, INT_EMB), lambda i: (0, 0)),
        ],
        out_specs=pl.BlockSpec((BE, HW), lambda i: (i, 0)),
        out_shape=jax.ShapeDtypeStruct((N_EDGES, HW), jnp.float32),
        compiler_params=pltpu.CompilerParams(
            dimension_semantics=("parallel",)),
    )(ea_t, w1t, b1r)

    zero = jnp.zeros((N_PAD, HW), jnp.float32)
    acc, deg = _scatter_kernel(h, idx2d, zero)
    deg2d = deg.reshape(NW, N_NODES).T

    out = pl.pallas_call(
        _out_body,
        grid=(N_NODES // BN,),
        in_specs=[
            pl.BlockSpec((NC, BN, HW), lambda i: (0, i, 0)),
            pl.BlockSpec((BN, NW), lambda i: (i, 0)),
            pl.BlockSpec((BN, HIDDEN), lambda i: (i, 0)),
            pl.BlockSpec((INT_EMB, HIDDEN), lambda i: (0, 0)),
            pl.BlockSpec((1, HIDDEN), lambda i: (0, 0)),
        ],
        out_specs=pl.BlockSpec((BN, HIDDEN), lambda i: (i, 0)),
        out_shape=jax.ShapeDtypeStruct((N_NODES, HIDDEN), jnp.float32),
        compiler_params=pltpu.CompilerParams(
            dimension_semantics=("parallel",)),
    )(acc, deg2d, x, w2t, b2r)
    return out


# speedup vs baseline: 1.8500x; 1.0509x over previous
import dataclasses
import functools

import jax
import jax.numpy as jnp
import numpy as np
from jax import lax
from jax.experimental import pallas as pl
from jax.experimental.pallas import tpu as pltpu
from jax.experimental.pallas import tpu_sc as plsc

NUM_RADIAL = 16
HIDDEN = 256
INT_EMB = 128
N_NODES = 10000
N_EDGES = 160000

HW = INT_EMB
NC, NS = 2, 16
NW = NC * NS
CHUNK = 128
N_CHUNKS = N_EDGES // CHUNK
CPW = 40
PAIRS = CPW // 2
N_PAD = 10240

BE = 32000
BN = 2000

_ZERO_INIT = np.zeros((N_PAD, HW), np.float32)


def _h_body(eat_ref, w1t_ref, b1_ref, h_ref):
    a = jax.lax.dot_general(
        eat_ref[...], w1t_ref[...], (((0,), (0,)), ((), ())),
        preferred_element_type=jnp.float32)
    a = a + b1_ref[...]
    h_ref[...] = a * jax.nn.sigmoid(a)


def _out_body(acc_ref, deg_ref, x_ref, w2t_ref, b2_ref, o_ref):
    nh = acc_ref[0] + acc_ref[1]
    deg = jnp.sum(deg_ref[...], axis=1, keepdims=True)
    o_ref[...] = (x_ref[...]
                  + jnp.dot(nh, w2t_ref[...], preferred_element_type=jnp.float32)
                  + deg * b2_ref[...])


_vmesh = plsc.VectorSubcoreMesh(core_axis_name="c", subcore_axis_name="s")

_sc_params = pltpu.CompilerParams()
if "needs_layout_passes" in pltpu.CompilerParams.__dataclass_fields__:
    _sc_params = dataclasses.replace(_sc_params, needs_layout_passes=False)


@functools.partial(
    pl.kernel,
    out_type=(
        jax.ShapeDtypeStruct((NC, N_NODES, HW), jnp.float32),
        jax.ShapeDtypeStruct((NW * N_NODES,), jnp.float32),
    ),
    mesh=_vmesh,
    compiler_params=_sc_params,
    scratch_types=[
        pltpu.VMEM((CPW, CHUNK), jnp.int32),
        pltpu.VMEM((CHUNK, HW), jnp.float32),
        pltpu.VMEM((CHUNK, HW), jnp.float32),
        pltpu.VMEM((N_PAD,), jnp.float32),
        pltpu.VMEM_SHARED((N_PAD, HW), jnp.float32),
        pltpu.SemaphoreType.DMA,
        pltpu.SemaphoreType.DMA,
    ],
)
def _scatter_kernel(h_hbm, idx_hbm, zero_hbm, out_hbm, deg_hbm,
                    idx_v, h_a, h_b, deg_v, acc_sh, sem_a, sem_b):
    c = lax.axis_index("c")
    s = lax.axis_index("s")
    wid = c * NS + s

    pltpu.sync_copy(zero_hbm.at[pl.ds(s * 640, 640)],
                    acc_sh.at[pl.ds(s * 640, 640)])

    zeros16 = jnp.zeros((16,), jnp.float32)
    @pl.loop(0, N_PAD // 16)
    def _(i):
        deg_v[pl.ds(i * 16, 16)] = zeros16

    plsc.subcore_barrier()

    base_chunk = wid * CPW
    e_base = base_chunk * CHUNK
    np_live = jnp.minimum(PAIRS, (N_CHUNKS - base_chunk) // 2)
    pltpu.sync_copy(idx_hbm.at[pl.ds(base_chunk, CPW)], idx_v)

    ones16 = jnp.ones((16,), jnp.float32)

    def deg_update(j):
        @pl.loop(0, CHUNK // 16)
        def _(k):
            idx16 = idx_v[j, pl.ds(k * 16, 16)]
            plsc.addupdate_scatter(deg_v, [idx16], ones16)

    n_live = 2 * np_live

    def load(j, buf, sem):
        pltpu.make_async_copy(
            h_hbm.at[pl.ds(e_base + j * CHUNK, CHUNK)], buf, sem).start()

    def drain_load(buf, sem):
        pltpu.make_async_copy(h_hbm.at[pl.ds(0, CHUNK)], buf, sem).wait()

    @pl.when(n_live > 0)
    def _():
        load(0, h_a, sem_a)

    @pl.loop(0, PAIRS)
    def _(t):
        c0 = 2 * t
        c1 = c0 + 1

        @pl.when(c0 < n_live)
        def _():
            @pl.when(c1 < n_live)
            def _():
                load(c1, h_b, sem_b)
            drain_load(h_a, sem_a)
            sc_a = pltpu.async_copy(h_a, acc_sh.at[idx_v.at[c0]], sem_a,
                                    add=True)
            deg_update(c0)
            sc_a.wait()

            @pl.when(c1 < n_live)
            def _():
                drain_load(h_b, sem_b)

                @pl.when(c0 + 2 < n_live)
                def _():
                    load(c0 + 2, h_a, sem_a)

                sc_b = pltpu.async_copy(h_b, acc_sh.at[idx_v.at[c1]], sem_b,
                                        add=True)
                deg_update(c1)
                sc_b.wait()

    pltpu.sync_copy(deg_v.at[pl.ds(0, N_NODES)],
                    deg_hbm.at[pl.ds(wid * N_NODES, N_NODES)])

    plsc.subcore_barrier()

    @pl.when(s < 10)
    def _():
        pltpu.sync_copy(acc_sh.at[pl.ds(s * 1000, 1000)],
                        out_hbm.at[c].at[pl.ds(s * 1000, 1000)])


def kernel(x, edge_index, edge_attr, W1, b1, W2, b2):
    col = edge_index[1].astype(jnp.int32)
    col_pad = jnp.concatenate(
        [col, jnp.zeros((NW * CPW * CHUNK - N_EDGES,), jnp.int32)])
    idx2d = col_pad.reshape(NW * CPW, CHUNK)

    w1t = W1.T
    b1r = b1.reshape(1, INT_EMB)
    w2t = W2.T
    b2r = b2.reshape(1, HIDDEN)

    ea_t = edge_attr.T
    h = pl.pallas_call(
        _h_body,
        grid=(N_EDGES // BE,),
        in_specs=[
            pl.BlockSpec((NUM_RADIAL, BE), lambda i: (0, i)),
            pl.BlockSpec((NUM_RADIAL, INT_EMB), lambda i: (0, 0)),
            pl.BlockSpec((1, INT_EMB), lambda i: (0, 0)),
        ],
        out_specs=pl.BlockSpec((BE, HW), lambda i: (i, 0)),
        out_shape=jax.ShapeDtypeStruct((N_EDGES, HW), jnp.float32),
        compiler_params=pltpu.CompilerParams(
            dimension_semantics=("parallel",)),
    )(ea_t, w1t, b1r)

    zero = jnp.zeros((N_PAD, HW), jnp.float32)
    acc, deg = _scatter_kernel(h, idx2d, zero)
    deg2d = deg.reshape(NW, N_NODES).T

    out = pl.pallas_call(
        _out_body,
        grid=(N_NODES // BN,),
        in_specs=[
            pl.BlockSpec((NC, BN, HW), lambda i: (0, i, 0)),
            pl.BlockSpec((BN, NW), lambda i: (i, 0)),
            pl.BlockSpec((BN, HIDDEN), lambda i: (i, 0)),
            pl.BlockSpec((INT_EMB, HIDDEN), lambda i: (0, 0)),
            pl.BlockSpec((1, HIDDEN), lambda i: (0, 0)),
        ],
        out_specs=pl.BlockSpec((BN, HIDDEN), lambda i: (i, 0)),
        out_shape=jax.ShapeDtypeStruct((N_NODES, HIDDEN), jnp.float32),
        compiler_params=pltpu.CompilerParams(
            dimension_semantics=("parallel",)),
    )(acc, deg2d, x, w2t, b2r)
    return out
